# baseline (speedup 1.0000x reference)
import functools

import jax
import jax.numpy as jnp
from jax import lax
from jax.experimental import pallas as pl
from jax.experimental.pallas import tpu as pltpu
from jax.experimental.pallas import tpu_sc as plsc

N = 100000
E = 1600000
G = 1024
F_XS, F_XT, F_E, F_U, F_OUT = 10, 5, 10, 10, 5

PAD = 8
NC, NS = 2, 16
NW = NC * NS
EW = E // NW
CHUNK = 1000

BN = 2048
NP = 102400


def _project_nodes_body(xst_ref, xtt_ref, w1_ref, ps_ref, pt_ref):
    w = w1_ref[...]
    dn = (((0,), (0,)), ((), ()))
    ps = lax.dot_general(xst_ref[...], w[0:F_XS], dn,
                         preferred_element_type=jnp.float32)
    pt = lax.dot_general(xtt_ref[...], w[F_XS:F_XS + F_XT], dn,
                         preferred_element_type=jnp.float32)
    z = jnp.zeros((BN, PAD - F_OUT), jnp.float32)
    ps_ref[...] = jnp.concatenate([ps, z], axis=1)
    pt_ref[...] = jnp.concatenate([pt, z], axis=1)


def _project_nodes(x_st, x_tt, w1):
    x_st = jnp.pad(x_st, ((0, 0), (0, NP - N)))
    x_tt = jnp.pad(x_tt, ((0, 0), (0, NP - N)))
    return pl.pallas_call(
        _project_nodes_body,
        grid=(NP // BN,),
        in_specs=[
            pl.BlockSpec((F_XS, BN), lambda i: (0, i)),
            pl.BlockSpec((F_XT, BN), lambda i: (0, i)),
            pl.BlockSpec(w1.shape, lambda i: (0, 0)),
        ],
        out_specs=[
            pl.BlockSpec((BN, PAD), lambda i: (i, 0)),
            pl.BlockSpec((BN, PAD), lambda i: (i, 0)),
        ],
        out_shape=[
            jax.ShapeDtypeStruct((NP, PAD), jnp.float32),
            jax.ShapeDtypeStruct((NP, PAD), jnp.float32),
        ],
    )(x_st, x_tt, w1)


def _project_globals_body(ut_ref, w1_ref, b1_ref, pu_ref):
    w = w1_ref[...]
    dn = (((0,), (0,)), ((), ()))
    pu = lax.dot_general(ut_ref[...], w[F_XS + F_XT + F_E:], dn,
                         preferred_element_type=jnp.float32) + b1_ref[...]
    z = jnp.zeros((G, PAD - F_OUT), jnp.float32)
    pu_ref[...] = jnp.concatenate([pu, z], axis=1)


def _project_globals(ut, w1, b1):
    return pl.pallas_call(
        _project_globals_body,
        out_shape=jax.ShapeDtypeStruct((G, PAD), jnp.float32),
    )(ut, w1, b1.reshape(1, F_OUT))


def _sc_gather_body(ps_hbm, pt_hbm, pu_hbm, ei_hbm, be_hbm,
                    gs_hbm, gt_hbm, gu_hbm,
                    src_v, tgt_v, be_v, rs_v, rt_v, ru_v,
                    sem_s, sem_t, sem_u):
    wid = lax.axis_index("s") * NC + lax.axis_index("c")

    def chunk_body(ci, carry):
        base = wid * EW + ci * CHUNK
        pltpu.sync_copy(ei_hbm.at[0, pl.ds(base, CHUNK)], src_v)
        pltpu.sync_copy(ei_hbm.at[1, pl.ds(base, CHUNK)], tgt_v)
        pltpu.sync_copy(be_hbm.at[pl.ds(base, CHUNK)], be_v)
        cp_s = pltpu.async_copy(ps_hbm.at[src_v], rs_v, sem_s)
        cp_t = pltpu.async_copy(pt_hbm.at[tgt_v], rt_v, sem_t)
        cp_u = pltpu.async_copy(pu_hbm.at[be_v], ru_v, sem_u)
        cp_s.wait()
        cp_t.wait()
        cp_u.wait()
        pltpu.sync_copy(rs_v, gs_hbm.at[pl.ds(base, CHUNK)])
        pltpu.sync_copy(rt_v, gt_hbm.at[pl.ds(base, CHUNK)])
        pltpu.sync_copy(ru_v, gu_hbm.at[pl.ds(base, CHUNK)])
        return carry

    lax.fori_loop(0, EW // CHUNK, chunk_body, 0)


def _sc_gather(ps, pt, pu, edge_index, be):
    kern = functools.partial(
        pl.kernel,
        out_type=(
            jax.ShapeDtypeStruct((E, PAD), jnp.float32),
            jax.ShapeDtypeStruct((E, PAD), jnp.float32),
            jax.ShapeDtypeStruct((E, PAD), jnp.float32),
        ),
        mesh=plsc.VectorSubcoreMesh(core_axis_name="c", subcore_axis_name="s"),
        compiler_params=pltpu.CompilerParams(use_tc_tiling_on_sc=False),
        scratch_types=[
            pltpu.VMEM((CHUNK,), jnp.int32),
            pltpu.VMEM((CHUNK,), jnp.int32),
            pltpu.VMEM((CHUNK,), jnp.int32),
            pltpu.VMEM((CHUNK, PAD), jnp.float32),
            pltpu.VMEM((CHUNK, PAD), jnp.float32),
            pltpu.VMEM((CHUNK, PAD), jnp.float32),
            pltpu.SemaphoreType.DMA,
            pltpu.SemaphoreType.DMA,
            pltpu.SemaphoreType.DMA,
        ],
    )(_sc_gather_body)
    return kern(ps, pt, pu, edge_index, be)


R = 64
ROWS = E // R
BR = 200


def _epilogue_body(gs_ref, gt_ref, gu_ref, ea_ref, sel_ref, exp_ref,
                   bd2_ref, b2_ref, o_ref):
    g = gs_ref[...] + gt_ref[...] + gu_ref[...]
    hs = jnp.dot(g, sel_ref[...], preferred_element_type=jnp.float32)
    ea = ea_ref[...]
    d = jnp.dot(ea[0], exp_ref[0], preferred_element_type=jnp.float32)
    for k in range(1, F_E):
        d = d + jnp.dot(ea[k], exp_ref[k],
                        preferred_element_type=jnp.float32)
    h = hs + d
    h = jnp.where(h > 0, h, 0.1 * h)
    o_ref[...] = jnp.dot(h, bd2_ref[...],
                         preferred_element_type=jnp.float32) + b2_ref[...]


def _epilogue(gs, gt, gu, edge_attr_t, w1e, w2, b2):
    gs_p = gs.reshape(ROWS, R * PAD)
    gt_p = gt.reshape(ROWS, R * PAD)
    gu_p = gu.reshape(ROWS, R * PAD)
    ea_p = edge_attr_t.reshape(F_E, ROWS, R)
    eye = jnp.eye(R, dtype=jnp.float32)
    p85 = jnp.zeros((PAD, F_OUT), jnp.float32).at[:F_OUT, :].set(
        jnp.eye(F_OUT, dtype=jnp.float32))
    sel = jnp.kron(eye, p85)
    exp = jnp.einsum("jl,kf->kjlf", eye, w1e).reshape(F_E, R, R * F_OUT)
    bd2 = jnp.kron(eye, w2)
    b2_t = jnp.tile(b2, R).reshape(1, R * F_OUT)
    out_p = pl.pallas_call(
        _epilogue_body,
        grid=(ROWS // BR,),
        in_specs=[
            pl.BlockSpec((BR, R * PAD), lambda i: (i, 0)),
            pl.BlockSpec((BR, R * PAD), lambda i: (i, 0)),
            pl.BlockSpec((BR, R * PAD), lambda i: (i, 0)),
            pl.BlockSpec((F_E, BR, R), lambda i: (0, i, 0)),
            pl.BlockSpec((R * PAD, R * F_OUT), lambda i: (0, 0)),
            pl.BlockSpec((F_E, R, R * F_OUT), lambda i: (0, 0, 0)),
            pl.BlockSpec((R * F_OUT, R * F_OUT), lambda i: (0, 0)),
            pl.BlockSpec((1, R * F_OUT), lambda i: (0, 0)),
        ],
        out_specs=pl.BlockSpec((BR, R * F_OUT), lambda i: (i, 0)),
        out_shape=jax.ShapeDtypeStruct((ROWS, R * F_OUT), jnp.float32),
    )(gs_p, gt_p, gu_p, ea_p, sel, exp, bd2, b2_t)
    return out_p.reshape(E, F_OUT)


def kernel(x_s, x_t, edge_index, edge_attr, u, batch_e, W1, b1, W2, b2):
    w1e = W1[F_XS + F_XT:F_XS + F_XT + F_E]
    ps, pt = _project_nodes(x_s.T, x_t.T, W1)
    pu = _project_globals(u.T, W1, b1)
    gs, gt, gu = _sc_gather(ps, pt, pu, edge_index, batch_e)
    return _epilogue(gs, gt, gu, edge_attr.T, w1e, W2, b2)


# speedup vs baseline: 1.2549x; 1.2549x over previous
import functools

import jax
import jax.numpy as jnp
from jax import lax
from jax.experimental import pallas as pl
from jax.experimental.pallas import tpu as pltpu
from jax.experimental.pallas import tpu_sc as plsc

N = 100000
E = 1600000
G = 1024
F_XS, F_XT, F_E, F_U, F_OUT = 10, 5, 10, 10, 5

PAD = 8
NC, NS = 2, 16
NW = NC * NS
EW = E // NW
CHUNK = 1000
NCH = EW // CHUNK

BN = 2048
NP = 102400


def _project_nodes_body(xst_ref, xtt_ref, w1_ref, ps_ref, pt_ref):
    w = w1_ref[...]
    dn = (((0,), (0,)), ((), ()))
    ps = lax.dot_general(xst_ref[...], w[0:F_XS], dn,
                         preferred_element_type=jnp.float32)
    pt = lax.dot_general(xtt_ref[...], w[F_XS:F_XS + F_XT], dn,
                         preferred_element_type=jnp.float32)
    z = jnp.zeros((BN, PAD - F_OUT), jnp.float32)
    ps_ref[...] = jnp.concatenate([ps, z], axis=1)
    pt_ref[...] = jnp.concatenate([pt, z], axis=1)


def _project_nodes(x_st, x_tt, w1):
    x_st = jnp.pad(x_st, ((0, 0), (0, NP - N)))
    x_tt = jnp.pad(x_tt, ((0, 0), (0, NP - N)))
    return pl.pallas_call(
        _project_nodes_body,
        grid=(NP // BN,),
        in_specs=[
            pl.BlockSpec((F_XS, BN), lambda i: (0, i)),
            pl.BlockSpec((F_XT, BN), lambda i: (0, i)),
            pl.BlockSpec(w1.shape, lambda i: (0, 0)),
        ],
        out_specs=[
            pl.BlockSpec((BN, PAD), lambda i: (i, 0)),
            pl.BlockSpec((BN, PAD), lambda i: (i, 0)),
        ],
        out_shape=[
            jax.ShapeDtypeStruct((NP, PAD), jnp.float32),
            jax.ShapeDtypeStruct((NP, PAD), jnp.float32),
        ],
    )(x_st, x_tt, w1)


def _project_globals_body(ut_ref, w1_ref, b1_ref, pu_ref):
    w = w1_ref[...]
    dn = (((0,), (0,)), ((), ()))
    pu = lax.dot_general(ut_ref[...], w[F_XS + F_XT + F_E:], dn,
                         preferred_element_type=jnp.float32) + b1_ref[...]
    z = jnp.zeros((G, PAD - F_OUT), jnp.float32)
    pu_ref[...] = jnp.concatenate([pu, z], axis=1)


def _project_globals(ut, w1, b1):
    return pl.pallas_call(
        _project_globals_body,
        out_shape=jax.ShapeDtypeStruct((G, PAD), jnp.float32),
    )(ut, w1, b1.reshape(1, F_OUT))


def _sc_gather_body(ps_hbm, pt_hbm, pu_hbm, ei_hbm, be_hbm,
                    gs_hbm, gt_hbm, gu_hbm,
                    src_v, tgt_v, be_v, rs_v, rt_v, ru_v,
                    sem_i0, sem_i1, sem_g0, sem_g1, sem_w0, sem_w1):
    wid = lax.axis_index("s") * NC + lax.axis_index("c")
    base0 = wid * EW
    sem_i = (sem_i0, sem_i1)
    sem_g = (sem_g0, sem_g1)
    sem_w = (sem_w0, sem_w1)

    def issue_idx(b, base):
        pltpu.async_copy(ei_hbm.at[0, pl.ds(base, CHUNK)], src_v.at[b],
                         sem_i[b])
        pltpu.async_copy(ei_hbm.at[1, pl.ds(base, CHUNK)], tgt_v.at[b],
                         sem_i[b])
        pltpu.async_copy(be_hbm.at[pl.ds(base, CHUNK)], be_v.at[b],
                         sem_i[b])

    def wait_idx(b):
        for _ in range(3):
            pltpu.make_async_copy(be_hbm.at[pl.ds(0, CHUNK)], be_v.at[b],
                                  sem_i[b]).wait()

    def issue_gather(b):
        pltpu.async_copy(ps_hbm.at[src_v.at[b]], rs_v.at[b], sem_g[b])
        pltpu.async_copy(pt_hbm.at[tgt_v.at[b]], rt_v.at[b], sem_g[b])
        pltpu.async_copy(pu_hbm.at[be_v.at[b]], ru_v.at[b], sem_g[b])

    def wait_gather(b):
        for _ in range(3):
            pltpu.make_async_copy(ps_hbm.at[pl.ds(0, CHUNK)], rs_v.at[b],
                                  sem_g[b]).wait()

    def issue_wb(b, base):
        pltpu.async_copy(rs_v.at[b], gs_hbm.at[pl.ds(base, CHUNK)], sem_w[b])
        pltpu.async_copy(rt_v.at[b], gt_hbm.at[pl.ds(base, CHUNK)], sem_w[b])
        pltpu.async_copy(ru_v.at[b], gu_hbm.at[pl.ds(base, CHUNK)], sem_w[b])

    def wait_wb(b):
        for _ in range(3):
            pltpu.make_async_copy(rs_v.at[b], gs_hbm.at[pl.ds(0, CHUNK)],
                                  sem_w[b]).wait()

    issue_idx(0, base0)
    issue_idx(1, base0 + CHUNK)

    def pair_body(ci2, carry):
        for b in range(2):
            ci = 2 * ci2 + b
            base = base0 + ci * CHUNK
            wait_idx(b)

            @pl.when(ci >= 2)
            def _reclaim_buffer():
                wait_wb(b)

            issue_gather(b)

            @pl.when(ci >= 1)
            def _retire_prev():
                wait_gather(1 - b)
                issue_wb(1 - b, base - CHUNK)

                @pl.when(ci + 1 < NCH)
                def _prefetch():
                    issue_idx(1 - b, base + CHUNK)

        return carry

    lax.fori_loop(0, NCH // 2, pair_body, 0)
    wait_gather(1)
    issue_wb(1, base0 + EW - CHUNK)
    wait_wb(0)
    wait_wb(1)


def _sc_gather(ps, pt, pu, edge_index, be):
    kern = functools.partial(
        pl.kernel,
        out_type=(
            jax.ShapeDtypeStruct((E, PAD), jnp.float32),
            jax.ShapeDtypeStruct((E, PAD), jnp.float32),
            jax.ShapeDtypeStruct((E, PAD), jnp.float32),
        ),
        mesh=plsc.VectorSubcoreMesh(core_axis_name="c", subcore_axis_name="s"),
        compiler_params=pltpu.CompilerParams(use_tc_tiling_on_sc=False),
        scratch_types=[
            pltpu.VMEM((2, CHUNK), jnp.int32),
            pltpu.VMEM((2, CHUNK), jnp.int32),
            pltpu.VMEM((2, CHUNK), jnp.int32),
            pltpu.VMEM((2, CHUNK, PAD), jnp.float32),
            pltpu.VMEM((2, CHUNK, PAD), jnp.float32),
            pltpu.VMEM((2, CHUNK, PAD), jnp.float32),
            pltpu.SemaphoreType.DMA,
            pltpu.SemaphoreType.DMA,
            pltpu.SemaphoreType.DMA,
            pltpu.SemaphoreType.DMA,
            pltpu.SemaphoreType.DMA,
            pltpu.SemaphoreType.DMA,
        ],
    )(_sc_gather_body)
    return kern(ps, pt, pu, edge_index, be)


R = 64
ROWS = E // R
BR = 200


def _epilogue_body(gs_ref, gt_ref, gu_ref, ea_ref, sel_ref, bd1_ref,
                   bd2_ref, b2_ref, o_ref):
    g = gs_ref[...] + gt_ref[...] + gu_ref[...]
    hs = jnp.dot(g, sel_ref[...], preferred_element_type=jnp.float32)
    d = jnp.dot(ea_ref[...], bd1_ref[...], preferred_element_type=jnp.float32)
    h = hs + d
    h = jnp.where(h > 0, h, 0.1 * h)
    o_ref[...] = jnp.dot(h, bd2_ref[...],
                         preferred_element_type=jnp.float32) + b2_ref[...]


def _epilogue(gs, gt, gu, edge_attr, w1e, w2, b2):
    gs_p = gs.reshape(ROWS, R * PAD)
    gt_p = gt.reshape(ROWS, R * PAD)
    gu_p = gu.reshape(ROWS, R * PAD)
    ea_p = edge_attr.reshape(ROWS, R * F_E)
    eye = jnp.eye(R, dtype=jnp.float32)
    p85 = jnp.zeros((PAD, F_OUT), jnp.float32).at[:F_OUT, :].set(
        jnp.eye(F_OUT, dtype=jnp.float32))
    sel = jnp.kron(eye, p85)
    bd1 = jnp.kron(eye, w1e)
    bd2 = jnp.kron(eye, w2)
    b2_t = jnp.tile(b2, R).reshape(1, R * F_OUT)
    out_p = pl.pallas_call(
        _epilogue_body,
        grid=(ROWS // BR,),
        in_specs=[
            pl.BlockSpec((BR, R * PAD), lambda i: (i, 0)),
            pl.BlockSpec((BR, R * PAD), lambda i: (i, 0)),
            pl.BlockSpec((BR, R * PAD), lambda i: (i, 0)),
            pl.BlockSpec((BR, R * F_E), lambda i: (i, 0)),
            pl.BlockSpec((R * PAD, R * F_OUT), lambda i: (0, 0)),
            pl.BlockSpec((R * F_E, R * F_OUT), lambda i: (0, 0)),
            pl.BlockSpec((R * F_OUT, R * F_OUT), lambda i: (0, 0)),
            pl.BlockSpec((1, R * F_OUT), lambda i: (0, 0)),
        ],
        out_specs=pl.BlockSpec((BR, R * F_OUT), lambda i: (i, 0)),
        out_shape=jax.ShapeDtypeStruct((ROWS, R * F_OUT), jnp.float32),
    )(gs_p, gt_p, gu_p, ea_p, sel, bd1, bd2, b2_t)
    return out_p.reshape(E, F_OUT)


def kernel(x_s, x_t, edge_index, edge_attr, u, batch_e, W1, b1, W2, b2):
    w1e = W1[F_XS + F_XT:F_XS + F_XT + F_E]
    ps, pt = _project_nodes(x_s.T, x_t.T, W1)
    pu = _project_globals(u.T, W1, b1)
    gs, gt, gu = _sc_gather(ps, pt, pu, edge_index, batch_e)
    return _epilogue(gs, gt, gu, edge_attr, w1e, W2, b2)


# speedup vs baseline: 1.3425x; 1.0698x over previous
import functools

import jax
import jax.numpy as jnp
from jax import lax
from jax.experimental import pallas as pl
from jax.experimental.pallas import tpu as pltpu
from jax.experimental.pallas import tpu_sc as plsc

N = 100000
E = 1600000
G = 1024
F_XS, F_XT, F_E, F_U, F_OUT = 10, 5, 10, 10, 5

PAD = 8
NC, NS = 2, 16
NW = NC * NS
EW = E // NW
CHUNK = 1000
NCH = EW // CHUNK

BN = 2048
NP = 102400


def _project_nodes_body(xst_ref, xtt_ref, w1_ref, ps_ref, pt_ref):
    w = w1_ref[...]
    dn = (((0,), (0,)), ((), ()))
    ps = lax.dot_general(xst_ref[...], w[0:F_XS], dn,
                         preferred_element_type=jnp.float32)
    pt = lax.dot_general(xtt_ref[...], w[F_XS:F_XS + F_XT], dn,
                         preferred_element_type=jnp.float32)
    z = jnp.zeros((BN, PAD - F_OUT), jnp.float32)
    ps_ref[...] = jnp.concatenate([ps, z], axis=1)
    pt_ref[...] = jnp.concatenate([pt, z], axis=1)


def _project_nodes(x_st, x_tt, w1):
    x_st = jnp.pad(x_st, ((0, 0), (0, NP - N)))
    x_tt = jnp.pad(x_tt, ((0, 0), (0, NP - N)))
    return pl.pallas_call(
        _project_nodes_body,
        grid=(NP // BN,),
        in_specs=[
            pl.BlockSpec((F_XS, BN), lambda i: (0, i)),
            pl.BlockSpec((F_XT, BN), lambda i: (0, i)),
            pl.BlockSpec(w1.shape, lambda i: (0, 0)),
        ],
        out_specs=[
            pl.BlockSpec((BN, PAD), lambda i: (i, 0)),
            pl.BlockSpec((BN, PAD), lambda i: (i, 0)),
        ],
        out_shape=[
            jax.ShapeDtypeStruct((NP, PAD), jnp.float32),
            jax.ShapeDtypeStruct((NP, PAD), jnp.float32),
        ],
    )(x_st, x_tt, w1)


def _project_globals_body(ut_ref, w1_ref, b1_ref, pu_ref):
    w = w1_ref[...]
    dn = (((0,), (0,)), ((), ()))
    pu = lax.dot_general(ut_ref[...], w[F_XS + F_XT + F_E:], dn,
                         preferred_element_type=jnp.float32) + b1_ref[...]
    z = jnp.zeros((G, PAD - F_OUT), jnp.float32)
    pu_ref[...] = jnp.concatenate([pu, z], axis=1)


def _project_globals(ut, w1, b1):
    return pl.pallas_call(
        _project_globals_body,
        out_shape=jax.ShapeDtypeStruct((G, PAD), jnp.float32),
    )(ut, w1, b1.reshape(1, F_OUT))


def _sc_gather_body(ps_hbm, pt_hbm, pu_hbm, ei_hbm, be_hbm,
                    gs_hbm, gt_hbm, gu_hbm,
                    src_v, tgt_v, be_v, rs_v, rt_v, ru_v,
                    sem_i0, sem_i1, sem_g0, sem_g1, sem_w0, sem_w1):
    wid = lax.axis_index("s") * NC + lax.axis_index("c")
    base0 = wid * EW
    sem_i = (sem_i0, sem_i1)
    sem_g = (sem_g0, sem_g1)
    sem_w = (sem_w0, sem_w1)

    def issue_idx(b, base):
        pltpu.async_copy(ei_hbm.at[0, pl.ds(base, CHUNK)], src_v.at[b],
                         sem_i[b])
        pltpu.async_copy(ei_hbm.at[1, pl.ds(base, CHUNK)], tgt_v.at[b],
                         sem_i[b])
        pltpu.async_copy(be_hbm.at[pl.ds(base, CHUNK)], be_v.at[b],
                         sem_i[b])

    def wait_idx(b):
        for _ in range(3):
            pltpu.make_async_copy(be_hbm.at[pl.ds(0, CHUNK)], be_v.at[b],
                                  sem_i[b]).wait()

    def issue_gather(b):
        pltpu.async_copy(ps_hbm.at[src_v.at[b]], rs_v.at[b], sem_g[b])
        pltpu.async_copy(pt_hbm.at[tgt_v.at[b]], rt_v.at[b], sem_g[b])
        pltpu.async_copy(pu_hbm.at[be_v.at[b]], ru_v.at[b], sem_g[b])

    def wait_gather(b):
        for _ in range(3):
            pltpu.make_async_copy(ps_hbm.at[pl.ds(0, CHUNK)], rs_v.at[b],
                                  sem_g[b]).wait()

    def issue_wb(b, base):
        pltpu.async_copy(rs_v.at[b], gs_hbm.at[pl.ds(base, CHUNK)], sem_w[b])
        pltpu.async_copy(rt_v.at[b], gt_hbm.at[pl.ds(base, CHUNK)], sem_w[b])
        pltpu.async_copy(ru_v.at[b], gu_hbm.at[pl.ds(base, CHUNK)], sem_w[b])

    def wait_wb(b):
        for _ in range(3):
            pltpu.make_async_copy(rs_v.at[b], gs_hbm.at[pl.ds(0, CHUNK)],
                                  sem_w[b]).wait()

    issue_idx(0, base0)
    issue_idx(1, base0 + CHUNK)

    def pair_body(ci2, carry):
        for b in range(2):
            ci = 2 * ci2 + b
            base = base0 + ci * CHUNK
            wait_idx(b)

            @pl.when(ci >= 2)
            def _reclaim_buffer():
                wait_wb(b)

            issue_gather(b)

            @pl.when(ci >= 1)
            def _retire_prev():
                wait_gather(1 - b)
                issue_wb(1 - b, base - CHUNK)

                @pl.when(ci + 1 < NCH)
                def _prefetch():
                    issue_idx(1 - b, base + CHUNK)

        return carry

    lax.fori_loop(0, NCH // 2, pair_body, 0)
    wait_gather(1)
    issue_wb(1, base0 + EW - CHUNK)
    wait_wb(0)
    wait_wb(1)


def _sc_gather(ps, pt, pu, edge_index, be):
    kern = functools.partial(
        pl.kernel,
        out_type=(
            jax.ShapeDtypeStruct((E, PAD), jnp.float32),
            jax.ShapeDtypeStruct((E, PAD), jnp.float32),
            jax.ShapeDtypeStruct((E, PAD), jnp.float32),
        ),
        mesh=plsc.VectorSubcoreMesh(core_axis_name="c", subcore_axis_name="s"),
        compiler_params=pltpu.CompilerParams(use_tc_tiling_on_sc=False),
        scratch_types=[
            pltpu.VMEM((2, CHUNK), jnp.int32),
            pltpu.VMEM((2, CHUNK), jnp.int32),
            pltpu.VMEM((2, CHUNK), jnp.int32),
            pltpu.VMEM((2, CHUNK, PAD), jnp.float32),
            pltpu.VMEM((2, CHUNK, PAD), jnp.float32),
            pltpu.VMEM((2, CHUNK, PAD), jnp.float32),
            pltpu.SemaphoreType.DMA,
            pltpu.SemaphoreType.DMA,
            pltpu.SemaphoreType.DMA,
            pltpu.SemaphoreType.DMA,
            pltpu.SemaphoreType.DMA,
            pltpu.SemaphoreType.DMA,
        ],
    )(_sc_gather_body)
    return kern(ps, pt, pu, edge_index, be)


R = 64
ROWS = E // R
BR = 200


def _epilogue_body(gs_ref, gt_ref, gu_ref, ea_ref, sel_ref, bd1_ref,
                   bd2_ref, b2_ref, o_ref):
    g = (gs_ref[...] + gt_ref[...] + gu_ref[...]).reshape(BR, R * PAD)
    hs = jnp.dot(g, sel_ref[...], preferred_element_type=jnp.float32)
    d = jnp.dot(ea_ref[...], bd1_ref[...], preferred_element_type=jnp.float32)
    h = hs + d
    h = jnp.where(h > 0, h, 0.1 * h)
    o_ref[...] = jnp.dot(h, bd2_ref[...],
                         preferred_element_type=jnp.float32) + b2_ref[...]


def _epilogue(gs, gt, gu, edge_attr, w1e, w2, b2):
    gs_p = gs.reshape(E * PAD)
    gt_p = gt.reshape(E * PAD)
    gu_p = gu.reshape(E * PAD)
    ea_p = edge_attr.reshape(ROWS, R * F_E)
    eye = jnp.eye(R, dtype=jnp.float32)
    p85 = jnp.zeros((PAD, F_OUT), jnp.float32).at[:F_OUT, :].set(
        jnp.eye(F_OUT, dtype=jnp.float32))
    sel = jnp.kron(eye, p85)
    bd1 = jnp.kron(eye, w1e)
    bd2 = jnp.kron(eye, w2)
    b2_t = jnp.tile(b2, R).reshape(1, R * F_OUT)
    out_p = pl.pallas_call(
        _epilogue_body,
        grid=(ROWS // BR,),
        in_specs=[
            pl.BlockSpec((BR * R * PAD,), lambda i: (i,)),
            pl.BlockSpec((BR * R * PAD,), lambda i: (i,)),
            pl.BlockSpec((BR * R * PAD,), lambda i: (i,)),
            pl.BlockSpec((BR, R * F_E), lambda i: (i, 0)),
            pl.BlockSpec((R * PAD, R * F_OUT), lambda i: (0, 0)),
            pl.BlockSpec((R * F_E, R * F_OUT), lambda i: (0, 0)),
            pl.BlockSpec((R * F_OUT, R * F_OUT), lambda i: (0, 0)),
            pl.BlockSpec((1, R * F_OUT), lambda i: (0, 0)),
        ],
        out_specs=pl.BlockSpec((BR, R * F_OUT), lambda i: (i, 0)),
        out_shape=jax.ShapeDtypeStruct((ROWS, R * F_OUT), jnp.float32),
    )(gs_p, gt_p, gu_p, ea_p, sel, bd1, bd2, b2_t)
    return out_p.reshape(E, F_OUT)


def kernel(x_s, x_t, edge_index, edge_attr, u, batch_e, W1, b1, W2, b2):
    w1e = W1[F_XS + F_XT:F_XS + F_XT + F_E]
    ps, pt = _project_nodes(x_s.T, x_t.T, W1)
    pu = _project_globals(u.T, W1, b1)
    gs, gt, gu = _sc_gather(ps, pt, pu, edge_index, batch_e)
    return _epilogue(gs, gt, gu, edge_attr, w1e, W2, b2)


# speedup vs baseline: 1.3842x; 1.0310x over previous
import functools

import jax
import jax.numpy as jnp
from jax import lax
from jax.experimental import pallas as pl
from jax.experimental.pallas import tpu as pltpu
from jax.experimental.pallas import tpu_sc as plsc

N = 100000
E = 1600000
G = 1024
F_XS, F_XT, F_E, F_U, F_OUT = 10, 5, 10, 10, 5

PAD = 8
NC, NS = 2, 16
NW = NC * NS
EW = E // NW
CHUNK = 1000
NCH = EW // CHUNK

BN = 2048
NP = 102400


def _project_nodes_body(xst_ref, xtt_ref, w1_ref, ps_ref, pt_ref):
    w = w1_ref[...]
    dn = (((0,), (0,)), ((), ()))
    ps = lax.dot_general(xst_ref[...], w[0:F_XS], dn,
                         preferred_element_type=jnp.float32)
    pt = lax.dot_general(xtt_ref[...], w[F_XS:F_XS + F_XT], dn,
                         preferred_element_type=jnp.float32)
    z = jnp.zeros((BN, PAD - F_OUT), jnp.float32)
    ps_ref[...] = jnp.concatenate([ps, z], axis=1)
    pt_ref[...] = jnp.concatenate([pt, z], axis=1)


def _project_nodes(x_st, x_tt, w1):
    x_st = jnp.pad(x_st, ((0, 0), (0, NP - N)))
    x_tt = jnp.pad(x_tt, ((0, 0), (0, NP - N)))
    return pl.pallas_call(
        _project_nodes_body,
        grid=(NP // BN,),
        in_specs=[
            pl.BlockSpec((F_XS, BN), lambda i: (0, i)),
            pl.BlockSpec((F_XT, BN), lambda i: (0, i)),
            pl.BlockSpec(w1.shape, lambda i: (0, 0)),
        ],
        out_specs=[
            pl.BlockSpec((BN, PAD), lambda i: (i, 0)),
            pl.BlockSpec((BN, PAD), lambda i: (i, 0)),
        ],
        out_shape=[
            jax.ShapeDtypeStruct((NP, PAD), jnp.float32),
            jax.ShapeDtypeStruct((NP, PAD), jnp.float32),
        ],
    )(x_st, x_tt, w1)


def _project_globals_body(ut_ref, w1_ref, b1_ref, pu_ref):
    w = w1_ref[...]
    dn = (((0,), (0,)), ((), ()))
    pu = lax.dot_general(ut_ref[...], w[F_XS + F_XT + F_E:], dn,
                         preferred_element_type=jnp.float32) + b1_ref[...]
    z = jnp.zeros((G, PAD - F_OUT), jnp.float32)
    pu_ref[...] = jnp.concatenate([pu, z], axis=1)


def _project_globals(ut, w1, b1):
    return pl.pallas_call(
        _project_globals_body,
        out_shape=jax.ShapeDtypeStruct((G, PAD), jnp.float32),
    )(ut, w1, b1.reshape(1, F_OUT))


def _sc_gather_body(ps_hbm, pt_hbm, pu_hbm, ei_hbm, be_hbm,
                    gs_hbm, gt_hbm, gu_hbm,
                    src_v, tgt_v, be_v, rs_v, rt_v, ru_v,
                    sem_i0, sem_i1, sem_g0, sem_g1, sem_w0, sem_w1):
    wid = lax.axis_index("s") * NC + lax.axis_index("c")
    base0 = wid * EW
    sem_i = (sem_i0, sem_i1)
    sem_g = (sem_g0, sem_g1)
    sem_w = (sem_w0, sem_w1)

    def issue_idx(b, base):
        pltpu.async_copy(ei_hbm.at[0, pl.ds(base, CHUNK)], src_v.at[b],
                         sem_i[b])
        pltpu.async_copy(ei_hbm.at[1, pl.ds(base, CHUNK)], tgt_v.at[b],
                         sem_i[b])
        pltpu.async_copy(be_hbm.at[pl.ds(base, CHUNK)], be_v.at[b],
                         sem_i[b])

    def wait_idx(b):
        for _ in range(3):
            pltpu.make_async_copy(be_hbm.at[pl.ds(0, CHUNK)], be_v.at[b],
                                  sem_i[b]).wait()

    def issue_gather(b):
        pltpu.async_copy(ps_hbm.at[src_v.at[b]], rs_v.at[b], sem_g[b])
        pltpu.async_copy(pt_hbm.at[tgt_v.at[b]], rt_v.at[b], sem_g[b])
        pltpu.async_copy(pu_hbm.at[be_v.at[b]], ru_v.at[b], sem_g[b])

    def wait_gather(b):
        for _ in range(3):
            pltpu.make_async_copy(ps_hbm.at[pl.ds(0, CHUNK)], rs_v.at[b],
                                  sem_g[b]).wait()

    def issue_wb(b, base):
        pltpu.async_copy(rs_v.at[b], gs_hbm.at[pl.ds(base, CHUNK)], sem_w[b])
        pltpu.async_copy(rt_v.at[b], gt_hbm.at[pl.ds(base, CHUNK)], sem_w[b])
        pltpu.async_copy(ru_v.at[b], gu_hbm.at[pl.ds(base, CHUNK)], sem_w[b])

    def wait_wb(b):
        for _ in range(3):
            pltpu.make_async_copy(rs_v.at[b], gs_hbm.at[pl.ds(0, CHUNK)],
                                  sem_w[b]).wait()

    issue_idx(0, base0)
    issue_idx(1, base0 + CHUNK)

    def pair_body(ci2, carry):
        for b in range(2):
            ci = 2 * ci2 + b
            base = base0 + ci * CHUNK
            wait_idx(b)

            @pl.when(ci >= 2)
            def _reclaim_buffer():
                wait_wb(b)

            issue_gather(b)

            @pl.when(ci >= 1)
            def _retire_prev():
                wait_gather(1 - b)
                issue_wb(1 - b, base - CHUNK)

                @pl.when(ci + 1 < NCH)
                def _prefetch():
                    issue_idx(1 - b, base + CHUNK)

        return carry

    lax.fori_loop(0, NCH // 2, pair_body, 0)
    wait_gather(1)
    issue_wb(1, base0 + EW - CHUNK)
    wait_wb(0)
    wait_wb(1)


def _sc_gather(ps, pt, pu, edge_index, be):
    kern = functools.partial(
        pl.kernel,
        out_type=(
            jax.ShapeDtypeStruct((E, PAD), jnp.float32),
            jax.ShapeDtypeStruct((E, PAD), jnp.float32),
            jax.ShapeDtypeStruct((E, PAD), jnp.float32),
        ),
        mesh=plsc.VectorSubcoreMesh(core_axis_name="c", subcore_axis_name="s"),
        compiler_params=pltpu.CompilerParams(use_tc_tiling_on_sc=False),
        scratch_types=[
            pltpu.VMEM((2, CHUNK), jnp.int32),
            pltpu.VMEM((2, CHUNK), jnp.int32),
            pltpu.VMEM((2, CHUNK), jnp.int32),
            pltpu.VMEM((2, CHUNK, PAD), jnp.float32),
            pltpu.VMEM((2, CHUNK, PAD), jnp.float32),
            pltpu.VMEM((2, CHUNK, PAD), jnp.float32),
            pltpu.SemaphoreType.DMA,
            pltpu.SemaphoreType.DMA,
            pltpu.SemaphoreType.DMA,
            pltpu.SemaphoreType.DMA,
            pltpu.SemaphoreType.DMA,
            pltpu.SemaphoreType.DMA,
        ],
    )(_sc_gather_body)
    return kern(ps, pt, pu, edge_index, be)


R = 64
ROWS = E // R
BR = 1000


def _epilogue_body(gs_ref, gt_ref, gu_ref, ea_ref, sel_ref, bd1_ref,
                   bd2_ref, b2_ref, o_ref):
    g = (gs_ref[...] + gt_ref[...] + gu_ref[...]).reshape(BR, R * PAD)
    hs = jnp.dot(g, sel_ref[...], preferred_element_type=jnp.float32)
    d = jnp.dot(ea_ref[...], bd1_ref[...], preferred_element_type=jnp.float32)
    h = hs + d
    h = jnp.where(h > 0, h, 0.1 * h)
    o_ref[...] = jnp.dot(h, bd2_ref[...],
                         preferred_element_type=jnp.float32) + b2_ref[...]


def _epilogue(gs, gt, gu, edge_attr, w1e, w2, b2):
    gs_p = gs.reshape(E * PAD)
    gt_p = gt.reshape(E * PAD)
    gu_p = gu.reshape(E * PAD)
    ea_p = edge_attr.reshape(ROWS, R * F_E)
    eye = jnp.eye(R, dtype=jnp.float32)
    p85 = jnp.zeros((PAD, F_OUT), jnp.float32).at[:F_OUT, :].set(
        jnp.eye(F_OUT, dtype=jnp.float32))
    sel = jnp.kron(eye, p85)
    bd1 = jnp.kron(eye, w1e)
    bd2 = jnp.kron(eye, w2)
    b2_t = jnp.tile(b2, R).reshape(1, R * F_OUT)
    out_p = pl.pallas_call(
        _epilogue_body,
        grid=(ROWS // BR,),
        in_specs=[
            pl.BlockSpec((BR * R * PAD,), lambda i: (i,)),
            pl.BlockSpec((BR * R * PAD,), lambda i: (i,)),
            pl.BlockSpec((BR * R * PAD,), lambda i: (i,)),
            pl.BlockSpec((BR, R * F_E), lambda i: (i, 0)),
            pl.BlockSpec((R * PAD, R * F_OUT), lambda i: (0, 0)),
            pl.BlockSpec((R * F_E, R * F_OUT), lambda i: (0, 0)),
            pl.BlockSpec((R * F_OUT, R * F_OUT), lambda i: (0, 0)),
            pl.BlockSpec((1, R * F_OUT), lambda i: (0, 0)),
        ],
        out_specs=pl.BlockSpec((BR, R * F_OUT), lambda i: (i, 0)),
        out_shape=jax.ShapeDtypeStruct((ROWS, R * F_OUT), jnp.float32),
    )(gs_p, gt_p, gu_p, ea_p, sel, bd1, bd2, b2_t)
    return out_p.reshape(E, F_OUT)


def kernel(x_s, x_t, edge_index, edge_attr, u, batch_e, W1, b1, W2, b2):
    w1e = W1[F_XS + F_XT:F_XS + F_XT + F_E]
    ps, pt = _project_nodes(x_s.T, x_t.T, W1)
    pu = _project_globals(u.T, W1, b1)
    gs, gt, gu = _sc_gather(ps, pt, pu, edge_index, batch_e)
    return _epilogue(gs, gt, gu, edge_attr, w1e, W2, b2)
